# baseline (speedup 1.0000x reference)
import functools

import jax
import jax.numpy as jnp
from jax import lax
from jax.experimental import pallas as pl
from jax.experimental.pallas import tpu as pltpu
from jax.experimental.pallas import tpu_sc as plsc


_TT = 1024
_CHUNK_ROWS = 8


def _tc_add_kernel(emb_ref, pos_ref, out_ref):
    out_ref[...] = emb_ref[...] + pos_ref[...]


def _tc_part(embeddings, pos_table, nb):
    Bn, Tn, Hn = embeddings.shape
    tt = _TT if Tn % _TT == 0 else Tn
    return pl.pallas_call(
        _tc_add_kernel,
        grid=(Tn // tt, nb),
        in_specs=[
            pl.BlockSpec((1, tt, Hn), lambda t, b: (b, t, 0)),
            pl.BlockSpec((tt, Hn), lambda t, b: (t, 0)),
        ],
        out_specs=pl.BlockSpec((1, tt, Hn), lambda t, b: (b, t, 0)),
        out_shape=jax.ShapeDtypeStruct((nb, Tn, Hn), embeddings.dtype),
    )(embeddings, pos_table)


def _sc_part(embeddings, pos_table, b_lo):
    Bn, Tn, Hn = embeddings.shape
    nb = Bn - b_lo
    info = plsc.get_sparse_core_info()
    nw = info.num_cores * info.num_subcores
    rows_w = Tn // nw
    chunks = rows_w // _CHUNK_ROWS
    mesh = plsc.VectorSubcoreMesh(core_axis_name="c", subcore_axis_name="s")

    @functools.partial(
        pl.kernel,
        mesh=mesh,
        out_type=jax.ShapeDtypeStruct((nb, Tn, Hn), jnp.float32),
        scratch_types=[
            pltpu.VMEM((2, _CHUNK_ROWS, Hn), jnp.float32),
            pltpu.VMEM((2, nb, _CHUNK_ROWS, Hn), jnp.float32),
            pltpu.SemaphoreType.DMA((2,)),
            pltpu.SemaphoreType.DMA((2,)),
        ],
    )
    def sc_k(emb_hbm, pos_hbm, out_hbm, pos_v, emb_v, lsem, ssem):
        wid = lax.axis_index("s") * info.num_cores + lax.axis_index("c")
        t0 = wid * rows_w

        def issue_loads(c, p):
            row = t0 + c * _CHUNK_ROWS
            cps = [pltpu.async_copy(
                pos_hbm.at[pl.ds(row, _CHUNK_ROWS)], pos_v.at[p], lsem.at[p])]
            for b in range(nb):
                cps.append(pltpu.async_copy(
                    emb_hbm.at[b_lo + b, pl.ds(row, _CHUNK_ROWS)],
                    emb_v.at[p, b], lsem.at[p]))
            return cps

        def issue_stores(c, p):
            row = t0 + c * _CHUNK_ROWS
            return [pltpu.async_copy(
                emb_v.at[p, b], out_hbm.at[b, pl.ds(row, _CHUNK_ROWS)],
                ssem.at[p]) for b in range(nb)]

        loads = {0: issue_loads(0, 0)}
        stores = {}
        for c in range(chunks):
            p = c % 2
            for cp in loads.pop(c):
                cp.wait()
            if c + 1 < chunks:
                if c - 1 in stores:
                    for cp in stores.pop(c - 1):
                        cp.wait()
                loads[c + 1] = issue_loads(c + 1, (c + 1) % 2)

            def body(j, _):
                r = j // (Hn // 16)
                col = (j % (Hn // 16)) * 16
                pv = pos_v[p, r, pl.ds(col, 16)]
                for b in range(nb):
                    emb_v[p, b, r, pl.ds(col, 16)] = (
                        emb_v[p, b, r, pl.ds(col, 16)] + pv)
                return 0

            lax.fori_loop(0, _CHUNK_ROWS * Hn // 16, body, 0)
            stores[c] = issue_stores(c, p)
        for cps in stores.values():
            for cp in cps:
                cp.wait()

    return sc_k(embeddings, pos_table)


def kernel(embeddings, pos_table):
    Bn, Tn, Hn = embeddings.shape
    b_lo = Bn - 1
    out_sc = _sc_part(embeddings, pos_table, b_lo)
    out_tc = _tc_part(embeddings, pos_table, b_lo)
    return jnp.concatenate([out_tc, out_sc], axis=0)


# speedup vs baseline: 1.4170x; 1.4170x over previous
import functools

import jax
import jax.numpy as jnp
from jax import lax
from jax.experimental import pallas as pl
from jax.experimental.pallas import tpu as pltpu
from jax.experimental.pallas import tpu_sc as plsc


_CHUNK_ROWS = 4
_NBUF = 8
_PF = 4


def kernel(embeddings, pos_table):
    Bn, Tn, Hn = embeddings.shape
    info = plsc.get_sparse_core_info()
    nw = info.num_cores * info.num_subcores
    rows_w = Tn // nw
    chunks = rows_w // _CHUNK_ROWS
    steps = chunks * Bn
    colshift = (Hn - 1).bit_length()
    mesh = plsc.VectorSubcoreMesh(core_axis_name="c", subcore_axis_name="s")

    @functools.partial(
        pl.kernel,
        mesh=mesh,
        out_type=jax.ShapeDtypeStruct((Bn, Tn, Hn), jnp.float32),
        scratch_types=[
            pltpu.VMEM((2, _CHUNK_ROWS, Hn), jnp.float32),
            pltpu.VMEM((_NBUF, _CHUNK_ROWS, Hn), jnp.float32),
            pltpu.SemaphoreType.DMA((2,)),
            pltpu.SemaphoreType.DMA((_NBUF,)),
            pltpu.SemaphoreType.DMA((_NBUF,)),
        ],
    )
    def sc_k(emb_hbm, pos_hbm, out_hbm, pos_v, emb_v, psem, lsem, ssem):
        wid = lax.axis_index("s") * info.num_cores + lax.axis_index("c")
        t0 = wid * rows_w

        def load_pos(c):
            return pltpu.async_copy(
                pos_hbm.at[pl.ds(t0 + c * _CHUNK_ROWS, _CHUNK_ROWS)],
                pos_v.at[c % 2], psem.at[c % 2])

        def load_emb(s):
            c, b = divmod(s, Bn)
            return pltpu.async_copy(
                emb_hbm.at[b, pl.ds(t0 + c * _CHUNK_ROWS, _CHUNK_ROWS)],
                emb_v.at[s % _NBUF], lsem.at[s % _NBUF])

        def store_out(s):
            c, b = divmod(s, Bn)
            return pltpu.async_copy(
                emb_v.at[s % _NBUF],
                out_hbm.at[b, pl.ds(t0 + c * _CHUNK_ROWS, _CHUNK_ROWS)],
                ssem.at[s % _NBUF])

        pos_cps = {c: load_pos(c) for c in range(min(2, chunks))}
        loads = {s: load_emb(s) for s in range(min(_PF, steps))}
        stores = {}
        for s in range(steps):
            c, b = divmod(s, Bn)
            if b == 0:
                pos_cps.pop(c).wait()
            loads.pop(s).wait()

            @plsc.parallel_loop(0, _CHUNK_ROWS * Hn, step=16, unroll=4)
            def _(i):
                r = i >> colshift
                col = pl.multiple_of(i - (r << colshift), 16)
                pv = pos_v[c % 2, r, pl.ds(col, 16)]
                emb_v[s % _NBUF, r, pl.ds(col, 16)] = (
                    emb_v[s % _NBUF, r, pl.ds(col, 16)] + pv)

            stores[s] = store_out(s)
            if b == Bn - 1 and c + 2 < chunks:
                pos_cps[c + 2] = load_pos(c + 2)
            ns = s + _PF
            if ns < steps:
                if ns - _NBUF >= 0:
                    stores.pop(ns - _NBUF).wait()
                loads[ns] = load_emb(ns)
        for cp in stores.values():
            cp.wait()

    return sc_k(embeddings, pos_table)


# speedup vs baseline: 1.4300x; 1.0092x over previous
import functools

import jax
import jax.numpy as jnp
from jax import lax
from jax.experimental import pallas as pl
from jax.experimental.pallas import tpu as pltpu
from jax.experimental.pallas import tpu_sc as plsc


_CHUNK_ROWS = 4
_NBUF = 12
_PF = 6


def kernel(embeddings, pos_table):
    Bn, Tn, Hn = embeddings.shape
    info = plsc.get_sparse_core_info()
    nw = info.num_cores * info.num_subcores
    rows_w = Tn // nw
    chunks = rows_w // _CHUNK_ROWS
    steps = chunks * Bn
    colshift = (Hn - 1).bit_length()
    mesh = plsc.VectorSubcoreMesh(core_axis_name="c", subcore_axis_name="s")

    @functools.partial(
        pl.kernel,
        mesh=mesh,
        out_type=jax.ShapeDtypeStruct((Bn, Tn, Hn), jnp.float32),
        scratch_types=[
            pltpu.VMEM((2, _CHUNK_ROWS, Hn), jnp.float32),
            pltpu.VMEM((_NBUF, _CHUNK_ROWS, Hn), jnp.float32),
            pltpu.SemaphoreType.DMA((2,)),
            pltpu.SemaphoreType.DMA((_NBUF,)),
            pltpu.SemaphoreType.DMA((_NBUF,)),
        ],
    )
    def sc_k(emb_hbm, pos_hbm, out_hbm, pos_v, emb_v, psem, lsem, ssem):
        wid = lax.axis_index("s") * info.num_cores + lax.axis_index("c")
        t0 = wid * rows_w

        def load_pos(c):
            return pltpu.async_copy(
                pos_hbm.at[pl.ds(t0 + c * _CHUNK_ROWS, _CHUNK_ROWS)],
                pos_v.at[c % 2], psem.at[c % 2])

        def load_emb(s):
            c, b = divmod(s, Bn)
            return pltpu.async_copy(
                emb_hbm.at[b, pl.ds(t0 + c * _CHUNK_ROWS, _CHUNK_ROWS)],
                emb_v.at[s % _NBUF], lsem.at[s % _NBUF])

        def store_out(s):
            c, b = divmod(s, Bn)
            return pltpu.async_copy(
                emb_v.at[s % _NBUF],
                out_hbm.at[b, pl.ds(t0 + c * _CHUNK_ROWS, _CHUNK_ROWS)],
                ssem.at[s % _NBUF])

        pos_cps = {c: load_pos(c) for c in range(min(2, chunks))}
        loads = {s: load_emb(s) for s in range(min(_PF, steps))}
        stores = {}
        for s in range(steps):
            c, b = divmod(s, Bn)
            if b == 0:
                pos_cps.pop(c).wait()
            loads.pop(s).wait()

            @plsc.parallel_loop(0, _CHUNK_ROWS * Hn, step=16, unroll=4)
            def _(i):
                r = i >> colshift
                col = pl.multiple_of(i - (r << colshift), 16)
                pv = pos_v[c % 2, r, pl.ds(col, 16)]
                emb_v[s % _NBUF, r, pl.ds(col, 16)] = (
                    emb_v[s % _NBUF, r, pl.ds(col, 16)] + pv)

            stores[s] = store_out(s)
            if b == Bn - 1 and c + 2 < chunks:
                pos_cps[c + 2] = load_pos(c + 2)
            ns = s + _PF
            if ns < steps:
                if ns - _NBUF >= 0:
                    stores.pop(ns - _NBUF).wait()
                loads[ns] = load_emb(ns)
        for cp in stores.values():
            cp.wait()

    return sc_k(embeddings, pos_table)
